# baseline (speedup 1.0000x reference)
import functools

import numpy as np
import jax
import jax.numpy as jnp
from jax import lax
from jax.experimental import pallas as pl
from jax.experimental.pallas import tpu as pltpu
from jax.experimental.pallas import tpu_sc as plsc

MAX_LEN = 200
EMB_DIM = 32
N_VOCAB = 27
BATCH = 4096

NC, NS = 2, 16
NW = NC * NS
ROWS_TOTAL = BATCH * MAX_LEN
ROWS_PER_W = ROWS_TOTAL // NW
GSIZE = 128
CHUNK = 1024
NG = CHUNK // GSIZE
NCHUNK = ROWS_PER_W // CHUNK

LG = 4
NLG = MAX_LEN // LG
LANES = LG * EMB_DIM
RB = 2048


def _pe_table() -> np.ndarray:
    pos = np.expand_dims(np.arange(MAX_LEN), 1)
    pe = pos / np.power(
        10000, 2 * np.expand_dims(np.arange(EMB_DIM) // 2, 0) / EMB_DIM)
    pe[:, 0::2] = np.sin(pe[:, 0::2])
    pe[:, 1::2] = np.cos(pe[:, 1::2])
    return pe.astype(np.float32)


def _fuse_body(emb_ref, pe_ref, o_ref):
    o_ref[...] = emb_ref[...][None, :, :] + pe_ref[...][:, None, :]


def _fused_table(embedding_weight, pe):
    t3 = pl.pallas_call(
        _fuse_body,
        out_shape=jax.ShapeDtypeStruct((MAX_LEN, N_VOCAB, EMB_DIM), jnp.float32),
    )(embedding_weight, pe)
    return t3.reshape(MAX_LEN * N_VOCAB, EMB_DIM)


def _idx_body(xt_ref, o_ref):
    lg = pl.program_id(0)
    lane_lk = lax.broadcasted_iota(jnp.int32, (BATCH // 32, 128), 1) & 3
    blk = xt_ref[0].T.reshape(BATCH // 32, 128)
    o_ref[...] = blk + (108 * lg + 27 * lane_lk)


@functools.cache
def _make_idx():
    return pl.pallas_call(
        _idx_body,
        grid=(NLG,),
        in_specs=[pl.BlockSpec((1, LG, BATCH), lambda lg: (lg, 0, 0))],
        out_specs=pl.BlockSpec((BATCH // 32, 128), lambda lg: (lg, 0)),
        out_shape=jax.ShapeDtypeStruct((ROWS_TOTAL // 128, 128), jnp.int32),
    )


def _sc_body(table_hbm, xp_hbm, out_hbm,
             idx_v0, idx_v1, idx_v2, rows_v0, rows_v1,
             xsem0, xsem1, xsem2, gsem0, gsem1, osem0, osem1):
    idx_v = (idx_v0, idx_v1, idx_v2)
    rows_v = (rows_v0, rows_v1)
    xsem = (xsem0, xsem1, xsem2)
    gsem = (gsem0, gsem1)
    osem = (osem0, osem1)
    wid = lax.axis_index("s") * NC + lax.axis_index("c")
    base = wid * ROWS_PER_W
    iota = lax.broadcasted_iota(jnp.int32, (16,), 0)
    lane27 = 27 * (iota & 3)

    def add_loff(c, xs):
        off = lane27 + 108 * ((base + c * CHUNK) >> 14)
        for g in range(CHUNK // 16):
            sl = pl.ds(g * 16, 16)
            idx_v[xs][sl] = idx_v[xs][sl] + off

    def x_copy(c, xs):
        return pltpu.make_async_copy(
            xp_hbm.at[pl.ds(base + c * CHUNK, CHUNK)], idx_v[xs], xsem[xs])

    def start_gathers(c, s, xs):
        for g in range(NG):
            pltpu.make_async_copy(
                table_hbm.at[idx_v[xs].at[pl.ds(g * GSIZE, GSIZE)]],
                rows_v[s].at[pl.ds(g * GSIZE, GSIZE)],
                gsem[s]).start()

    def drain_gathers(c, s, xs):
        for g in range(NG):
            pltpu.make_async_copy(
                table_hbm.at[idx_v[xs].at[pl.ds(g * GSIZE, GSIZE)]],
                rows_v[s].at[pl.ds(g * GSIZE, GSIZE)],
                gsem[s]).wait()

    def out_copy(c, s):
        return pltpu.make_async_copy(
            rows_v[s], out_hbm.at[pl.ds(base + c * CHUNK, CHUNK)], osem[s])

    def chunk_step(c, ms2, ms3, wait_out, prefetch):
        prev3 = (ms3 + 2) % 3
        if wait_out:
            out_copy(c - 2, ms2).wait()
        x_copy(c, ms3).wait()
        add_loff(c, ms3)
        start_gathers(c, ms2, ms3)
        drain_gathers(c - 1, 1 - ms2, prev3)
        if prefetch:
            x_copy(c + 2, prev3).start()
        out_copy(c - 1, 1 - ms2).start()

    x_copy(0, 0).start()
    x_copy(1, 1).start()
    x_copy(2, 2).start()
    x_copy(0, 0).wait()
    add_loff(0, 0)
    start_gathers(0, 0, 0)
    x_copy(1, 1).wait()
    add_loff(1, 1)
    start_gathers(1, 1, 1)
    drain_gathers(0, 0, 0)
    x_copy(3, 0).start()
    out_copy(0, 0).start()
    chunk_step(2, 0, 2, wait_out=True, prefetch=True)

    def six_body(p, carry):
        c0 = 6 * p + 3
        for k in range(6):
            c = c0 + k
            chunk_step(c, (3 + k) % 2, k % 3, wait_out=True, prefetch=True)
        return carry

    lax.fori_loop(0, 3, six_body, 0)

    chunk_step(21, 1, 0, wait_out=True, prefetch=True)
    chunk_step(22, 0, 1, wait_out=True, prefetch=True)
    chunk_step(23, 1, 2, wait_out=True, prefetch=False)
    chunk_step(24, 0, 0, wait_out=True, prefetch=False)
    drain_gathers(NCHUNK - 1, 0, (NCHUNK - 1) % 3)
    out_copy(NCHUNK - 1, 0).start()
    out_copy(NCHUNK - 2, 1).wait()
    out_copy(NCHUNK - 1, 0).wait()


@functools.cache
def _make_sc_gather():
    return pl.kernel(
        _sc_body,
        out_type=jax.ShapeDtypeStruct((ROWS_TOTAL, EMB_DIM), jnp.float32),
        mesh=plsc.VectorSubcoreMesh(
            core_axis_name="c", subcore_axis_name="s", num_cores=NC,
            num_subcores=NS),
        scratch_types=[
            pltpu.VMEM((CHUNK,), jnp.int32),
            pltpu.VMEM((CHUNK,), jnp.int32),
            pltpu.VMEM((CHUNK,), jnp.int32),
            pltpu.VMEM((CHUNK, EMB_DIM), jnp.float32),
            pltpu.VMEM((CHUNK, EMB_DIM), jnp.float32),
            pltpu.SemaphoreType.DMA,
            pltpu.SemaphoreType.DMA,
            pltpu.SemaphoreType.DMA,
            pltpu.SemaphoreType.DMA,
            pltpu.SemaphoreType.DMA,
            pltpu.SemaphoreType.DMA,
            pltpu.SemaphoreType.DMA,
        ],
        compiler_params=pltpu.CompilerParams(use_tc_tiling_on_sc=False),
    )


def _retile_body(lin_ref, o_ref):
    for i in range(RB // 128):
        o_ref[:, :, i * 128:(i + 1) * 128] = (
            lin_ref[i * 128:(i + 1) * 128, :].T.reshape(LG, EMB_DIM, 128))


@functools.cache
def _make_retile():
    return pl.pallas_call(
        _retile_body,
        grid=(NLG, BATCH // RB),
        in_specs=[pl.BlockSpec((RB, LANES),
                               lambda lg, bb: (lg * (BATCH // RB) + bb, 0))],
        out_specs=pl.BlockSpec((LG, EMB_DIM, RB), lambda lg, bb: (lg, 0, bb)),
        out_shape=jax.ShapeDtypeStruct((MAX_LEN, EMB_DIM, BATCH), jnp.float32),
    )


def kernel(x, embedding_weight):
    pe = jnp.asarray(_pe_table())
    table = _fused_table(embedding_weight, pe)
    xw = lax.bitcast_convert_type(
        x.astype(jnp.int8).reshape(BATCH, NLG, LG), jnp.int32)
    x_perm = (lax.bitcast_convert_type(xw.T, jnp.int8)
              .astype(jnp.int32)
              .reshape(ROWS_TOTAL))
    lin = _make_sc_gather()(table, x_perm)
    lin128 = lin.reshape(ROWS_TOTAL // LG, LANES)
    out_t = _make_retile()(lin128)
    return out_t.transpose(2, 0, 1)


# speedup vs baseline: 1.0302x; 1.0302x over previous
import functools

import numpy as np
import jax
import jax.numpy as jnp
from jax import lax
from jax.experimental import pallas as pl
from jax.experimental.pallas import tpu as pltpu
from jax.experimental.pallas import tpu_sc as plsc

MAX_LEN = 200
EMB_DIM = 32
N_VOCAB = 27
BATCH = 4096

NC, NS = 2, 16
NW = NC * NS
ROWS_TOTAL = BATCH * MAX_LEN
ROWS_PER_W = ROWS_TOTAL // NW
GSIZE = 128
CHUNK = 1024
NG = CHUNK // GSIZE
NCHUNK = ROWS_PER_W // CHUNK

LG = 4
NLG = MAX_LEN // LG
LANES = LG * EMB_DIM
RB = 2048


def _pe_table() -> np.ndarray:
    pos = np.expand_dims(np.arange(MAX_LEN), 1)
    pe = pos / np.power(
        10000, 2 * np.expand_dims(np.arange(EMB_DIM) // 2, 0) / EMB_DIM)
    pe[:, 0::2] = np.sin(pe[:, 0::2])
    pe[:, 1::2] = np.cos(pe[:, 1::2])
    return pe.astype(np.float32)


def _fuse_body(emb_ref, pe_ref, o_ref):
    o_ref[...] = emb_ref[...][None, :, :] + pe_ref[...][:, None, :]


def _fused_table(embedding_weight, pe):
    t3 = pl.pallas_call(
        _fuse_body,
        out_shape=jax.ShapeDtypeStruct((MAX_LEN, N_VOCAB, EMB_DIM), jnp.float32),
    )(embedding_weight, pe)
    return t3.reshape(MAX_LEN * N_VOCAB, EMB_DIM)


def _idx_body(xt_ref, o_ref):
    lg = pl.program_id(0)
    lane_lk = lax.broadcasted_iota(jnp.int32, (BATCH // 32, 128), 1) & 3
    blk = xt_ref[0].T.reshape(BATCH // 32, 128)
    o_ref[...] = blk + (108 * lg + 27 * lane_lk)


@functools.cache
def _make_idx():
    return pl.pallas_call(
        _idx_body,
        grid=(NLG,),
        in_specs=[pl.BlockSpec((1, LG, BATCH), lambda lg: (lg, 0, 0))],
        out_specs=pl.BlockSpec((BATCH // 32, 128), lambda lg: (lg, 0)),
        out_shape=jax.ShapeDtypeStruct((ROWS_TOTAL // 128, 128), jnp.int32),
    )


def _sc_body(table_hbm, xp_hbm, out_hbm,
             idx_v0, idx_v1, idx_v2, rows_v0, rows_v1,
             xsem0, xsem1, xsem2, gsem0, gsem1, osem0, osem1):
    idx_v = (idx_v0, idx_v1, idx_v2)
    rows_v = (rows_v0, rows_v1)
    xsem = (xsem0, xsem1, xsem2)
    gsem = (gsem0, gsem1)
    osem = (osem0, osem1)
    wid = lax.axis_index("s") * NC + lax.axis_index("c")
    base = wid * ROWS_PER_W
    iota = lax.broadcasted_iota(jnp.int32, (16,), 0)
    lane27 = 27 * (iota & 3)

    def add_loff(c, xs):
        off = lane27 + 108 * ((base + c * CHUNK) >> 14)
        for g in range(CHUNK // 16):
            sl = pl.ds(g * 16, 16)
            idx_v[xs][sl] = idx_v[xs][sl] + off

    def x_copy(c, xs):
        return pltpu.make_async_copy(
            xp_hbm.at[pl.ds(base + c * CHUNK, CHUNK)], idx_v[xs], xsem[xs])

    def start_gathers(c, s, xs):
        for g in range(NG):
            pltpu.make_async_copy(
                table_hbm.at[idx_v[xs].at[pl.ds(g * GSIZE, GSIZE)]],
                rows_v[s].at[pl.ds(g * GSIZE, GSIZE)],
                gsem[s]).start()

    def drain_gathers(c, s, xs):
        for g in range(NG):
            pltpu.make_async_copy(
                table_hbm.at[idx_v[xs].at[pl.ds(g * GSIZE, GSIZE)]],
                rows_v[s].at[pl.ds(g * GSIZE, GSIZE)],
                gsem[s]).wait()

    def out_copy(c, s):
        return pltpu.make_async_copy(
            rows_v[s], out_hbm.at[pl.ds(base + c * CHUNK, CHUNK)], osem[s])

    def chunk_step(c, ms2, ms3, wait_out, prefetch):
        prev3 = (ms3 + 2) % 3
        if wait_out:
            out_copy(c - 2, ms2).wait()
        x_copy(c, ms3).wait()
        add_loff(c, ms3)
        start_gathers(c, ms2, ms3)
        drain_gathers(c - 1, 1 - ms2, prev3)
        if prefetch:
            x_copy(c + 2, prev3).start()
        out_copy(c - 1, 1 - ms2).start()

    x_copy(0, 0).start()
    x_copy(1, 1).start()
    x_copy(2, 2).start()
    x_copy(0, 0).wait()
    add_loff(0, 0)
    start_gathers(0, 0, 0)
    x_copy(1, 1).wait()
    add_loff(1, 1)
    start_gathers(1, 1, 1)
    drain_gathers(0, 0, 0)
    x_copy(3, 0).start()
    out_copy(0, 0).start()
    chunk_step(2, 0, 2, wait_out=True, prefetch=True)

    def six_body(p, carry):
        c0 = 6 * p + 3
        for k in range(6):
            c = c0 + k
            chunk_step(c, (3 + k) % 2, k % 3, wait_out=True, prefetch=True)
        return carry

    lax.fori_loop(0, 3, six_body, 0)

    chunk_step(21, 1, 0, wait_out=True, prefetch=True)
    chunk_step(22, 0, 1, wait_out=True, prefetch=True)
    chunk_step(23, 1, 2, wait_out=True, prefetch=False)
    chunk_step(24, 0, 0, wait_out=True, prefetch=False)
    drain_gathers(NCHUNK - 1, 0, (NCHUNK - 1) % 3)
    out_copy(NCHUNK - 1, 0).start()
    out_copy(NCHUNK - 2, 1).wait()
    out_copy(NCHUNK - 1, 0).wait()


@functools.cache
def _make_sc_gather():
    return pl.kernel(
        _sc_body,
        out_type=jax.ShapeDtypeStruct((ROWS_TOTAL, EMB_DIM), jnp.float32),
        mesh=plsc.VectorSubcoreMesh(
            core_axis_name="c", subcore_axis_name="s", num_cores=NC,
            num_subcores=NS),
        scratch_types=[
            pltpu.VMEM((CHUNK,), jnp.int32),
            pltpu.VMEM((CHUNK,), jnp.int32),
            pltpu.VMEM((CHUNK,), jnp.int32),
            pltpu.VMEM((CHUNK, EMB_DIM), jnp.float32),
            pltpu.VMEM((CHUNK, EMB_DIM), jnp.float32),
            pltpu.SemaphoreType.DMA,
            pltpu.SemaphoreType.DMA,
            pltpu.SemaphoreType.DMA,
            pltpu.SemaphoreType.DMA,
            pltpu.SemaphoreType.DMA,
            pltpu.SemaphoreType.DMA,
            pltpu.SemaphoreType.DMA,
        ],
        compiler_params=pltpu.CompilerParams(use_tc_tiling_on_sc=False),
    )


def _retile_body(lin_ref, o_ref):
    for i in range(RB // 128):
        o_ref[:, :, i * 128:(i + 1) * 128] = (
            lin_ref[i * 128:(i + 1) * 128, :].T.reshape(LG, EMB_DIM, 128))


@functools.cache
def _make_retile():
    return pl.pallas_call(
        _retile_body,
        grid=(NLG, BATCH // RB),
        in_specs=[pl.BlockSpec((RB, LANES),
                               lambda lg, bb: (lg * (BATCH // RB) + bb, 0))],
        out_specs=pl.BlockSpec((LG, EMB_DIM, RB), lambda lg, bb: (lg, 0, bb)),
        out_shape=jax.ShapeDtypeStruct((MAX_LEN, EMB_DIM, BATCH), jnp.float32),
    )


def kernel(x, embedding_weight):
    pe = jnp.asarray(_pe_table())
    table = _fused_table(embedding_weight, pe)
    x_perm = (x.astype(jnp.int32).T
              .reshape(NLG, LG, BATCH)
              .transpose(0, 2, 1)
              .reshape(ROWS_TOTAL))
    lin = _make_sc_gather()(table, x_perm)
    lin128 = lin.reshape(ROWS_TOTAL // LG, LANES)
    out_t = _make_retile()(lin128)
    return out_t.transpose(2, 0, 1)
